# baseline (speedup 1.0000x reference)
import jax
import jax.numpy as jnp
from jax import lax
from jax.experimental import pallas as pl
from jax.experimental.pallas import tpu as pltpu
from jax.experimental.pallas import tpu_sc as plsc

N = 10000
E = 320000
D = 128

NC = 2
NS = 16
L = 16

EDGES_PER_TILE = E // NC // NS
CHUNK = 80
NCHUNK = EDGES_PER_TILE // CHUNK
N_PAD = 10240
ROWS_PER_TILE = N_PAD // NS
ROW_CHUNK = CHUNK
NROWC = ROWS_PER_TILE // ROW_CHUNK
NBUF = 2


def _sc_body(x, src, dst, w, out, acc,
             idxv, dstv, dsts, wv, rows_g, rows_s, sem_g, sem_s):
    c = lax.axis_index("c")
    s = lax.axis_index("s")

    def zrow(i, carry):
        for j in range(D // L):
            rows_s[0, i, pl.ds(j * L, L)] = jnp.zeros((L,), jnp.float32)
        return carry

    lax.fori_loop(0, ROW_CHUNK, zrow, 0)

    def zcopy(k, carry):
        pltpu.sync_copy(
            rows_s.at[0],
            acc.at[pl.ds(s * ROWS_PER_TILE + k * ROW_CHUNK, ROW_CHUNK)])
        return carry

    lax.fori_loop(0, NROWC, zcopy, 0)
    plsc.subcore_barrier()

    base = (c * NS + s) * EDGES_PER_TILE

    def issue_io(t, b):
        eb = base + t * CHUNK
        pltpu.sync_copy(src.at[pl.ds(eb, CHUNK)], idxv.at[b])
        pltpu.sync_copy(dst.at[pl.ds(eb, CHUNK)], dstv.at[b])
        pltpu.sync_copy(w.at[pl.ds(eb, CHUNK)], wv.at[b])
        pltpu.async_copy(x.at[idxv.at[b]], rows_g.at[b], sem_g.at[b])

    for b in range(NBUF):
        issue_io(b, b)

    def outer(tt, carry):
        for b in range(NBUF):
            t = tt * NBUF + b

            @pl.when(t < NCHUNK)
            def _phase():
                pltpu.make_async_copy(
                    x.at[idxv.at[b]], rows_g.at[b], sem_g.at[b]).wait()

                @pl.when(t >= NBUF)
                def _():
                    pltpu.make_async_copy(
                        rows_s.at[b], acc.at[dsts.at[b]], sem_s.at[b]).wait()

                def srow(g, icarry):
                    w16 = wv[b, pl.ds(g * L, L)]
                    di = dstv[b, pl.ds(g * L, L)]
                    dsts[b, pl.ds(g * L, L)] = di
                    for k in range(L):
                        i = g * L + k
                        wi = w16[k]
                        for j in range(D // L):
                            sl = pl.ds(j * L, L)
                            rows_s[b, i, sl] = rows_g[b, i, sl] * wi
                    return icarry

                lax.fori_loop(0, CHUNK // L, srow, 0)

                pltpu.async_copy(
                    rows_s.at[b], acc.at[dsts.at[b]], sem_s.at[b], add=True)

                @pl.when(t + NBUF < NCHUNK)
                def _():
                    issue_io(t + NBUF, b)

            _ = _phase
        return carry

    lax.fori_loop(0, (NCHUNK + NBUF - 1) // NBUF, outer, 0)
    for b in range(NBUF):
        pltpu.make_async_copy(
            rows_s.at[b], acc.at[dsts.at[b]], sem_s.at[b]).wait()
    plsc.subcore_barrier()

    def wout(k, carry):
        r0 = s * ROWS_PER_TILE + k * ROW_CHUNK
        pltpu.sync_copy(acc.at[pl.ds(r0, ROW_CHUNK)], rows_g.at[0])
        pltpu.sync_copy(rows_g.at[0], out.at[c, pl.ds(r0, ROW_CHUNK)])
        return carry

    lax.fori_loop(0, NROWC, wout, 0)


def _sum_body(p_ref, o_ref):
    o_ref[...] = p_ref[0] + p_ref[1]


_SUM_BR = 400


def kernel(x, edge_index, edge_weight):
    src = edge_index[1].astype(jnp.int32)
    dst = edge_index[0].astype(jnp.int32)
    w = edge_weight.astype(jnp.float32)

    mesh = plsc.VectorSubcoreMesh(core_axis_name="c", subcore_axis_name="s")
    partials = pl.kernel(
        _sc_body,
        out_type=jax.ShapeDtypeStruct((NC, N_PAD, D), jnp.float32),
        mesh=mesh,
        scratch_types=[
            pltpu.VMEM_SHARED((N_PAD, D), jnp.float32),
            pltpu.VMEM((NBUF, CHUNK), jnp.int32),
            pltpu.VMEM((NBUF, CHUNK), jnp.int32),
            pltpu.VMEM((NBUF, CHUNK), jnp.int32),
            pltpu.VMEM((NBUF, CHUNK), jnp.float32),
            pltpu.VMEM((NBUF, CHUNK, D), jnp.float32),
            pltpu.VMEM((NBUF, CHUNK, D), jnp.float32),
            pltpu.SemaphoreType.DMA((NBUF,)),
            pltpu.SemaphoreType.DMA((NBUF,)),
        ],
    )(x, src, dst, w)

    out = pl.pallas_call(
        _sum_body,
        out_shape=jax.ShapeDtypeStruct((N, D), jnp.float32),
        grid=(N // _SUM_BR,),
        in_specs=[pl.BlockSpec((NC, _SUM_BR, D), lambda i: (0, i, 0))],
        out_specs=pl.BlockSpec((_SUM_BR, D), lambda i: (i, 0)),
    )(partials)
    return out
